# baseline (speedup 1.0000x reference)
import functools

import jax
import jax.numpy as jnp
from jax import lax
from jax.experimental import pallas as pl
from jax.experimental.pallas import tpu as pltpu
from jax.experimental.pallas import tpu_sc as plsc

_N = 2048
_D = 1024
_L = 64
_K = 8
_F = 2048
_COEFF = 0.01

_BN = 256
_GRID = _N // _BN

_SC = plsc.get_sparse_core_info()
_NW = _SC.num_cores * _SC.num_subcores
_RPT = _N // _NW
_NEG = jnp.float32(-1e30)


def _logits_body(enc_ref, wenc_ref, benc_ref, out_ref):
    out_ref[...] = jnp.dot(enc_ref[...], wenc_ref[...],
                           preferred_element_type=jnp.float32) + benc_ref[...]


@functools.partial(
    pl.kernel,
    mesh=plsc.VectorSubcoreMesh(core_axis_name="c", subcore_axis_name="s"),
    out_type=[
        jax.ShapeDtypeStruct((_N, 16), jnp.float32),
        jax.ShapeDtypeStruct((_N, 16), jnp.int32),
    ],
    scratch_types=[
        pltpu.VMEM((_RPT, _L), jnp.float32),
        pltpu.VMEM((_RPT, 16), jnp.float32),
        pltpu.VMEM((_RPT, 16), jnp.int32),
    ],
)
def _sc_route(logits_hbm, probs_hbm, idx_hbm, lg_v, pr_v, ix_v):
    wid = lax.axis_index("s") * _SC.num_cores + lax.axis_index("c")
    base = wid * _RPT
    pltpu.sync_copy(logits_hbm.at[pl.ds(base, _RPT)], lg_v)

    iota16 = lax.iota(jnp.int32, 16)
    lane_ok = iota16 < _K

    def _gather16(v, idx):
        dn = lax.GatherDimensionNumbers(
            offset_dims=(), collapsed_slice_dims=(0,), start_index_map=(0,))
        return lax.gather(v, idx.reshape(16, 1), dn, (1,),
                          mode=lax.GatherScatterMode.PROMISE_IN_BOUNDS)

    def _allmax(v):
        for sh in (1, 2, 4, 8):
            v = jnp.maximum(v, _gather16(v, iota16 ^ sh))
        return v

    def _allmin_i(v):
        for sh in (1, 2, 4, 8):
            v = jnp.minimum(v, _gather16(v, iota16 ^ sh))
        return v

    def _allsum(v):
        for sh in (1, 2, 4, 8):
            v = v + _gather16(v, iota16 ^ sh)
        return v

    def _one_row(r):
        vs = [lg_v[r, pl.ds(g * 16, 16)] for g in range(4)]
        z = jnp.zeros((16,), jnp.float32)
        zi = jnp.zeros((16,), jnp.int32)
        cz = jnp.zeros((16,), jnp.float32)
        run = jnp.zeros((16,), jnp.float32)
        for k in range(_K):
            m = jnp.maximum(jnp.maximum(vs[0], vs[1]),
                            jnp.maximum(vs[2], vs[3]))
            glob = _allmax(m)
            cands = [jnp.where(vs[g] == glob, g * 16 + iota16, _L)
                     for g in range(4)]
            mc = jnp.minimum(jnp.minimum(cands[0], cands[1]),
                             jnp.minimum(cands[2], cands[3]))
            flat_sel = _allmin_i(mc)
            g_sel = lax.shift_right_logical(flat_sel, 4)
            lane_sel = flat_sel & 15
            hit_lane = iota16 == lane_sel
            vs = [jnp.where((g_sel == g) & hit_lane, _NEG, vs[g])
                  for g in range(4)]
            at_k = iota16 == k
            run = run + glob
            z = jnp.where(at_k, glob, z)
            cz = jnp.where(at_k, run, cz)
            zi = jnp.where(at_k, flat_sel, zi)

        rk = (iota16 + 1).astype(jnp.float32)
        cond = (1.0 + rk * z > cz) & lane_ok
        ksel = _allsum(jnp.where(cond, 1.0, 0.0))
        czsel = _gather16(cz, ksel.astype(jnp.int32) - 1)
        tau = (czsel - 1.0) / ksel
        pr = jnp.where(lane_ok, jnp.maximum(z - tau, 0.0), 0.0)
        pr_v[r, :] = pr
        ix_v[r, :] = zi

    def row(r, carry):
        _one_row(2 * r)
        _one_row(2 * r + 1)
        return carry

    lax.fori_loop(0, _RPT // 2, row, 0)
    pltpu.sync_copy(pr_v, probs_hbm.at[pl.ds(base, _RPT)])
    pltpu.sync_copy(ix_v, idx_hbm.at[pl.ds(base, _RPT)])


def _main_body(dec_ref, lab_ref, probs_ref, idx_ref, w1_ref, w3_ref, w2_ref,
               out_ref):
    i = pl.program_id(0)

    probs = probs_ref[...][:, :_K]
    idx = idx_ref[...]

    p_safe = jnp.where(probs > 0, probs, 1.0)
    ent_sum = -jnp.sum(probs * jnp.log(p_safe))

    iota_l = jax.lax.broadcasted_iota(jnp.int32, (_BN, _L), 1)
    u = jnp.dot(dec_ref[...], w3_ref[...], preferred_element_type=jnp.float32)
    lab = lab_ref[...]
    acc = -_COEFF * ent_sum
    for k in range(_K):
        oh = (iota_l == idx[:, k:k + 1]).astype(jnp.float32)
        w1row = jnp.dot(oh, w1_ref[...], preferred_element_type=jnp.float32)
        h = jnp.maximum(u + w1row, 0.0)
        y = jnp.dot(h, w2_ref[...], preferred_element_type=jnp.float32)
        dlt = y - lab
        lc = jnp.sum(dlt * dlt, axis=1) * (1.0 / _D)
        acc = acc + jnp.sum(probs[:, k] * lc)

    acc2d = acc.reshape(1, 1)
    out_ref[...] = jnp.where(i == 0, acc2d, out_ref[...] + acc2d)


def kernel(encoder_input, decoder_input, labels, W_enc, b_enc, W1, W3, W2):
    logits = pl.pallas_call(
        _logits_body,
        grid=(4,),
        in_specs=[
            pl.BlockSpec((_N // 4, _D), lambda i: (i, 0)),
            pl.BlockSpec((_D, _L), lambda i: (0, 0)),
            pl.BlockSpec((1, _L), lambda i: (0, 0)),
        ],
        out_specs=pl.BlockSpec((_N // 4, _L), lambda i: (i, 0)),
        out_shape=jax.ShapeDtypeStruct((_N, _L), jnp.float32),
    )(encoder_input, W_enc, b_enc.reshape(1, _L))

    probs16, idx16 = _sc_route(logits)

    out = pl.pallas_call(
        _main_body,
        grid=(_GRID,),
        in_specs=[
            pl.BlockSpec((_BN, _D), lambda i: (i, 0)),
            pl.BlockSpec((_BN, _D), lambda i: (i, 0)),
            pl.BlockSpec((_BN, 16), lambda i: (i, 0)),
            pl.BlockSpec((_BN, 16), lambda i: (i, 0)),
            pl.BlockSpec((_L, _F), lambda i: (0, 0)),
            pl.BlockSpec((_D, _F), lambda i: (0, 0)),
            pl.BlockSpec((_F, _D), lambda i: (0, 0)),
        ],
        out_specs=pl.BlockSpec((1, 1), lambda i: (0, 0)),
        out_shape=jax.ShapeDtypeStruct((1, 1), jnp.float32),
    )(decoder_input, labels, probs16, idx16, W1, W3, W2)
    return out[0, 0] / _N


# speedup vs baseline: 1.0224x; 1.0224x over previous
import functools

import jax
import jax.numpy as jnp
from jax import lax
from jax.experimental import pallas as pl
from jax.experimental.pallas import tpu as pltpu
from jax.experimental.pallas import tpu_sc as plsc

_N = 2048
_D = 1024
_L = 64
_K = 8
_F = 2048
_COEFF = 0.01

_BN = 256
_GRID = _N // _BN

_SC = plsc.get_sparse_core_info()
_NW = _SC.num_cores * _SC.num_subcores
_RPT = _N // _NW
_NEG = jnp.float32(-1e30)


def _logits_body(enc_ref, wenc_ref, benc_ref, out_ref):
    out_ref[...] = jnp.dot(enc_ref[...], wenc_ref[...],
                           preferred_element_type=jnp.float32) + benc_ref[...]


@functools.partial(
    pl.kernel,
    mesh=plsc.VectorSubcoreMesh(core_axis_name="c", subcore_axis_name="s"),
    out_type=[
        jax.ShapeDtypeStruct((_N, 16), jnp.float32),
        jax.ShapeDtypeStruct((_N, 16), jnp.int32),
    ],
    scratch_types=[
        pltpu.VMEM((_RPT, _L), jnp.float32),
        pltpu.VMEM((_RPT, 16), jnp.float32),
        pltpu.VMEM((_RPT, 16), jnp.int32),
    ],
)
def _sc_route(logits_hbm, probs_hbm, idx_hbm, lg_v, pr_v, ix_v):
    wid = lax.axis_index("s") * _SC.num_cores + lax.axis_index("c")
    base = wid * _RPT
    pltpu.sync_copy(logits_hbm.at[pl.ds(base, _RPT)], lg_v)

    iota16 = lax.iota(jnp.int32, 16)
    lane_ok = iota16 < _K

    def _gather16(v, idx):
        dn = lax.GatherDimensionNumbers(
            offset_dims=(), collapsed_slice_dims=(0,), start_index_map=(0,))
        return lax.gather(v, idx.reshape(16, 1), dn, (1,),
                          mode=lax.GatherScatterMode.PROMISE_IN_BOUNDS)

    def _allmax(v):
        for sh in (1, 2, 4, 8):
            v = jnp.maximum(v, _gather16(v, iota16 ^ sh))
        return v

    def _allmin_i(v):
        for sh in (1, 2, 4, 8):
            v = jnp.minimum(v, _gather16(v, iota16 ^ sh))
        return v

    def _allsum(v):
        for sh in (1, 2, 4, 8):
            v = v + _gather16(v, iota16 ^ sh)
        return v

    def _one_row(r):
        vs = [lg_v[r, pl.ds(g * 16, 16)] for g in range(4)]
        z = jnp.zeros((16,), jnp.float32)
        zi = jnp.zeros((16,), jnp.int32)
        cz = jnp.zeros((16,), jnp.float32)
        run = jnp.zeros((16,), jnp.float32)
        for k in range(_K):
            m = jnp.maximum(jnp.maximum(vs[0], vs[1]),
                            jnp.maximum(vs[2], vs[3]))
            glob = _allmax(m)
            cands = [jnp.where(vs[g] == glob, g * 16 + iota16, _L)
                     for g in range(4)]
            mc = jnp.minimum(jnp.minimum(cands[0], cands[1]),
                             jnp.minimum(cands[2], cands[3]))
            flat_sel = _allmin_i(mc)
            g_sel = lax.shift_right_logical(flat_sel, 4)
            lane_sel = flat_sel & 15
            hit_lane = iota16 == lane_sel
            vs = [jnp.where((g_sel == g) & hit_lane, _NEG, vs[g])
                  for g in range(4)]
            at_k = iota16 == k
            run = run + glob
            z = jnp.where(at_k, glob, z)
            cz = jnp.where(at_k, run, cz)
            zi = jnp.where(at_k, flat_sel, zi)

        rk = (iota16 + 1).astype(jnp.float32)
        cond = (1.0 + rk * z > cz) & lane_ok
        ksel = _allsum(jnp.where(cond, 1.0, 0.0))
        czsel = _gather16(cz, ksel.astype(jnp.int32) - 1)
        tau = (czsel - 1.0) / ksel
        pr = jnp.where(lane_ok, jnp.maximum(z - tau, 0.0), 0.0)
        pr_v[r, :] = pr
        ix_v[r, :] = zi

    def row(r, carry):
        _one_row(2 * r)
        _one_row(2 * r + 1)
        return carry

    lax.fori_loop(0, _RPT // 2, row, 0)
    pltpu.sync_copy(pr_v, probs_hbm.at[pl.ds(base, _RPT)])
    pltpu.sync_copy(ix_v, idx_hbm.at[pl.ds(base, _RPT)])


def _u_body(dec_ref, w3_ref, u_ref):
    u_ref[...] = jnp.dot(dec_ref[...], w3_ref[...],
                         preferred_element_type=jnp.float32)


def _main_body(u_ref, lab_ref, probs_ref, idx_ref, w1_ref, w2_ref,
               out_ref):
    i = pl.program_id(0)

    probs = probs_ref[...][:, :_K]
    idx = idx_ref[...]

    p_safe = jnp.where(probs > 0, probs, 1.0)
    ent_sum = -jnp.sum(probs * jnp.log(p_safe))

    iota_l = jax.lax.broadcasted_iota(jnp.int32, (_BN, _L), 1)
    u = u_ref[...]
    lab = lab_ref[...]
    acc = -_COEFF * ent_sum
    for k in range(_K):
        oh = (iota_l == idx[:, k:k + 1]).astype(jnp.float32)
        w1row = jnp.dot(oh, w1_ref[...], preferred_element_type=jnp.float32)
        h = jnp.maximum(u + w1row, 0.0)
        y = jnp.dot(h, w2_ref[...], preferred_element_type=jnp.float32)
        dlt = y - lab
        lc = jnp.sum(dlt * dlt, axis=1) * (1.0 / _D)
        acc = acc + jnp.sum(probs[:, k] * lc)

    acc2d = acc.reshape(1, 1)
    out_ref[...] = jnp.where(i == 0, acc2d, out_ref[...] + acc2d)


def kernel(encoder_input, decoder_input, labels, W_enc, b_enc, W1, W3, W2):
    logits = pl.pallas_call(
        _logits_body,
        grid=(4,),
        in_specs=[
            pl.BlockSpec((_N // 4, _D), lambda i: (i, 0)),
            pl.BlockSpec((_D, _L), lambda i: (0, 0)),
            pl.BlockSpec((1, _L), lambda i: (0, 0)),
        ],
        out_specs=pl.BlockSpec((_N // 4, _L), lambda i: (i, 0)),
        out_shape=jax.ShapeDtypeStruct((_N, _L), jnp.float32),
    )(encoder_input, W_enc, b_enc.reshape(1, _L))

    probs16, idx16 = _sc_route(logits)

    u_full = pl.pallas_call(
        _u_body,
        grid=(4,),
        in_specs=[
            pl.BlockSpec((_N // 4, _D), lambda i: (i, 0)),
            pl.BlockSpec((_D, _F), lambda i: (0, 0)),
        ],
        out_specs=pl.BlockSpec((_N // 4, _F), lambda i: (i, 0)),
        out_shape=jax.ShapeDtypeStruct((_N, _F), jnp.float32),
    )(decoder_input, W3)

    out = pl.pallas_call(
        _main_body,
        grid=(_GRID,),
        in_specs=[
            pl.BlockSpec((_BN, _F), lambda i: (i, 0)),
            pl.BlockSpec((_BN, _D), lambda i: (i, 0)),
            pl.BlockSpec((_BN, 16), lambda i: (i, 0)),
            pl.BlockSpec((_BN, 16), lambda i: (i, 0)),
            pl.BlockSpec((_L, _F), lambda i: (0, 0)),
            pl.BlockSpec((_F, _D), lambda i: (0, 0)),
        ],
        out_specs=pl.BlockSpec((1, 1), lambda i: (0, 0)),
        out_shape=jax.ShapeDtypeStruct((1, 1), jnp.float32),
    )(u_full, labels, probs16, idx16, W1, W2)
    return out[0, 0] / _N
